# baseline (speedup 1.0000x reference)
import functools
import math

import jax
import jax.numpy as jnp
from jax import lax
from jax.experimental import pallas as pl
from jax.experimental.pallas import tpu as pltpu
from jax.experimental.pallas import tpu_sc as plsc

D_MODEL = 1024
SCALE = math.sqrt(float(D_MODEL))
LANES = 16
VECS = D_MODEL // LANES

NC = 2
NS = 16
NW = NC * NS

B_TOT = 4 * 4096
B_PER_W = B_TOT // NW
C = 32
NCHUNK = B_PER_W // C
NBUF = 3

_mesh = plsc.VectorSubcoreMesh(core_axis_name="c", subcore_axis_name="s")


@functools.partial(
    pl.kernel,
    mesh=_mesh,
    out_type=jax.ShapeDtypeStruct((B_TOT, D_MODEL), jnp.float32),
    scratch_types=[
        pltpu.VMEM((NCHUNK, C), jnp.int32),
        pltpu.VMEM((C, D_MODEL), jnp.float32),
        pltpu.VMEM((C, D_MODEL), jnp.float32),
        pltpu.VMEM((C, D_MODEL), jnp.float32),
        pltpu.SemaphoreType.DMA,
        pltpu.SemaphoreType.DMA,
        pltpu.SemaphoreType.DMA,
        pltpu.SemaphoreType.DMA,
        pltpu.SemaphoreType.DMA,
        pltpu.SemaphoreType.DMA,
    ],
)
def _emb_lookup(x_hbm, lut_hbm, out_hbm, idx_v, b0, b1, b2,
                si0, si1, si2, so0, so1, so2):
    wid = lax.axis_index("s") * NC + lax.axis_index("c")
    base = wid * B_PER_W
    pltpu.sync_copy(x_hbm.at[wid], idx_v)
    scale = jnp.full((LANES,), SCALE, jnp.float32)

    bufs = [b0, b1, b2]
    sin = [si0, si1, si2]
    sout = [so0, so1, so2]

    def gather(g, b):
        return pltpu.async_copy(lut_hbm.at[idx_v.at[g]], bufs[b], sin[b])

    def outcopy(g, b):
        return pltpu.async_copy(
            bufs[b], out_hbm.at[pl.ds(base + g * C, C)], sout[b])

    def multiply(b):
        buf = bufs[b]

        def mul_rows(r2, _):
            r = r2 * 2
            for rr in range(2):
                for j in range(VECS):
                    sl = pl.ds(j * LANES, LANES)
                    buf[r + rr, sl] = buf[r + rr, sl] * scale
            return 0

        lax.fori_loop(0, C // 2, mul_rows, 0)

    copies_in = {0: gather(0, 0), 1: gather(1, 1)}
    copies_out = {}
    for g in range(NCHUNK):
        b = g % NBUF
        copies_in[g].wait()
        multiply(b)
        copies_out[g] = outcopy(g, b)
        if g + 2 < NCHUNK:
            if g - 1 >= 0:
                copies_out[g - 1].wait()
            copies_in[g + 2] = gather(g + 2, (g + 2) % NBUF)
    copies_out[NCHUNK - 2].wait()
    copies_out[NCHUNK - 1].wait()


def kernel(x, lut):
    xf = x.reshape(NW, NCHUNK, C)
    out = _emb_lookup(xf, lut)
    return out.reshape(4, 4096, D_MODEL)


# speedup vs baseline: 1.2279x; 1.2279x over previous
import functools
import math

import jax
import jax.numpy as jnp
from jax import lax
from jax.experimental import pallas as pl
from jax.experimental.pallas import tpu as pltpu
from jax.experimental.pallas import tpu_sc as plsc

D_MODEL = 1024
SCALE = math.sqrt(float(D_MODEL))
LANES = 16
VECS = D_MODEL // LANES

NC = 2
NS = 16
NW = NC * NS

B_TOT = 4 * 4096
B_PER_W = B_TOT // NW
C = 32
NCHUNK = B_PER_W // C
NBUF = 3

_mesh = plsc.VectorSubcoreMesh(core_axis_name="c", subcore_axis_name="s")


@functools.partial(
    pl.kernel,
    mesh=_mesh,
    out_type=jax.ShapeDtypeStruct((B_TOT, D_MODEL), jnp.float32),
    scratch_types=[
        pltpu.VMEM((NCHUNK, C), jnp.int32),
        pltpu.VMEM((C, D_MODEL), jnp.float32),
        pltpu.VMEM((C, D_MODEL), jnp.float32),
        pltpu.VMEM((C, D_MODEL), jnp.float32),
        pltpu.SemaphoreType.DMA,
        pltpu.SemaphoreType.DMA,
        pltpu.SemaphoreType.DMA,
        pltpu.SemaphoreType.DMA,
        pltpu.SemaphoreType.DMA,
        pltpu.SemaphoreType.DMA,
    ],
)
def _emb_lookup(x_hbm, lut_hbm, out_hbm, idx_v, b0, b1, b2,
                si0, si1, si2, so0, so1, so2):
    wid = lax.axis_index("s") * NC + lax.axis_index("c")
    base = wid * B_PER_W
    pltpu.sync_copy(x_hbm.at[wid], idx_v)
    scale = jnp.full((LANES,), SCALE, jnp.float32)

    bufs = [b0, b1, b2]
    sin = [si0, si1, si2]
    sout = [so0, so1, so2]

    def gather_start(g, b):
        return pltpu.async_copy(lut_hbm.at[idx_v.at[g]], bufs[b], sin[b])

    def gather_wait(b):
        pltpu.make_async_copy(lut_hbm.at[idx_v.at[0]], bufs[b], sin[b]).wait()

    def out_start(g, b):
        return pltpu.async_copy(
            bufs[b], out_hbm.at[pl.ds(base + g * C, C)], sout[b])

    def out_wait(b):
        pltpu.make_async_copy(
            bufs[b], out_hbm.at[pl.ds(base, C)], sout[b]).wait()

    def multiply(b):
        buf = bufs[b]

        def mul_row(r, _):
            for j in range(VECS):
                sl = pl.ds(j * LANES, LANES)
                buf[r, sl] = buf[r, sl] * scale
            return 0

        lax.fori_loop(0, C, mul_row, 0)

    gather_start(0, 0)
    gather_start(1, 1)
    gather_wait(0)
    multiply(0)
    out_start(0, 0)
    gather_start(2, 2)

    def loop_body(i, _):
        for bb in range(NBUF):
            b = (1 + bb) % NBUF
            g = 1 + i * NBUF + bb
            gather_wait(b)
            multiply(b)
            out_start(g, b)

            @pl.when(g + 2 < NCHUNK)
            def _():
                out_wait(bb)
                gather_start(g + 2, bb)

        return 0

    lax.fori_loop(0, (NCHUNK - 1) // NBUF, loop_body, 0)
    out_wait(1)
    out_wait(2)
    out_wait(0)


def kernel(x, lut):
    xf = x.reshape(NW, NCHUNK, C)
    out = _emb_lookup(xf, lut)
    return out.reshape(4, 4096, D_MODEL)
